# baseline (speedup 1.0000x reference)
import functools

import jax
import jax.numpy as jnp
from jax.experimental import pallas as pl
from jax.experimental.pallas import tpu as pltpu

_N = 10000
_BM = 80
_GM = _N // _BM
_STEPS = 2 * _GM
_NBUF = 12
_SPLITS = (0, _BM)


def _gcn_body(adj_hbm, x_ref, w0_ref, b0_ref, w1_ref, b1_ref, out_hbm,
              bufs, obuf, xw0_s, hw1_s, hw1bf_s, in_sems, out_sems):

    def adj_copies(g, slot):
        row = jax.lax.rem(g, _GM) * _BM
        return [
            pltpu.make_async_copy(
                adj_hbm.at[pl.ds(row + lo, hi - lo), :],
                bufs.at[slot, pl.ds(lo, hi - lo), :],
                in_sems.at[slot, p])
            for p, (lo, hi) in enumerate(zip(_SPLITS[:-1], _SPLITS[1:]))
        ]

    def adj_start(g, slot):
        for c in adj_copies(g, slot):
            c.start()

    def adj_wait(g, slot):
        for c in adj_copies(g, slot):
            c.wait()

    for k in range(_NBUF):
        adj_start(k, k)

    xw0_s[...] = jnp.dot(
        x_ref[...].astype(jnp.bfloat16),
        w0_ref[...].astype(jnp.bfloat16),
        preferred_element_type=jnp.float32,
    ).astype(jnp.bfloat16)
    w1b = w1_ref[...].astype(jnp.bfloat16)
    b0v = b0_ref[...]
    b1v = b1_ref[...]

    def refill(g, slot):
        @pl.when(g + _NBUF < _STEPS)
        def _():
            adj_start(g + _NBUF, slot)

    def phase0_step(i, carry):
        slot = jax.lax.rem(i, _NBUF)
        adj_wait(i, slot)
        a = bufs[slot].astype(jnp.bfloat16)
        acc = jnp.dot(a, xw0_s[...], preferred_element_type=jnp.float32)
        h = jnp.maximum(acc + b0v, 0.0)
        hw1_s[pl.ds(i * _BM, _BM), :] = jnp.dot(
            h.astype(jnp.bfloat16), w1b, preferred_element_type=jnp.float32)
        refill(i, slot)
        return carry

    jax.lax.fori_loop(0, _GM, phase0_step, 0, unroll=_NBUF)

    hw1bf_s[...] = hw1_s[...].astype(jnp.bfloat16)

    def out_copy(j, oslot):
        return pltpu.make_async_copy(
            obuf.at[oslot], out_hbm.at[pl.ds(j * _BM, _BM), :],
            out_sems.at[oslot])

    def phase1_step(j, carry):
        g = _GM + j
        slot = jax.lax.rem(g, _NBUF)
        adj_wait(g, slot)
        a = bufs[slot].astype(jnp.bfloat16)
        o = jnp.dot(a, hw1bf_s[...], preferred_element_type=jnp.float32) + b1v
        oslot = jax.lax.rem(j, 2)

        @pl.when(j >= 2)
        def _():
            out_copy(j - 2, oslot).wait()

        obuf[oslot] = o
        out_copy(j, oslot).start()
        refill(g, slot)
        return carry

    jax.lax.fori_loop(0, _GM, phase1_step, 0, unroll=_NBUF)

    out_copy(_GM - 2, jax.lax.rem(_GM - 2, 2)).wait()
    out_copy(_GM - 1, jax.lax.rem(_GM - 1, 2)).wait()


@functools.partial(jax.jit, donate_argnums=())
def kernel(x, adj, W0, b0, W1, b1):
    n, d_in = x.shape
    d_hid = W0.shape[1]
    d_out = W1.shape[1]
    b0r = b0.reshape(1, d_hid)
    b1r = b1.reshape(1, d_out)

    out = pl.pallas_call(
        _gcn_body,
        in_specs=[
            pl.BlockSpec(memory_space=pltpu.MemorySpace.HBM),
            pl.BlockSpec(memory_space=pltpu.MemorySpace.VMEM),
            pl.BlockSpec(memory_space=pltpu.MemorySpace.VMEM),
            pl.BlockSpec(memory_space=pltpu.MemorySpace.VMEM),
            pl.BlockSpec(memory_space=pltpu.MemorySpace.VMEM),
            pl.BlockSpec(memory_space=pltpu.MemorySpace.VMEM),
        ],
        out_specs=pl.BlockSpec(memory_space=pltpu.MemorySpace.HBM),
        out_shape=jax.ShapeDtypeStruct((n, d_out), jnp.float32),
        scratch_shapes=[
            pltpu.VMEM((_NBUF, _BM, _N), jnp.float32),
            pltpu.VMEM((2, _BM, 128), jnp.float32),
            pltpu.VMEM((_N, 128), jnp.bfloat16),
            pltpu.VMEM((_N, 128), jnp.float32),
            pltpu.VMEM((_N, 128), jnp.bfloat16),
            pltpu.SemaphoreType.DMA((_NBUF, len(_SPLITS) - 1)),
            pltpu.SemaphoreType.DMA((2,)),
        ],
    )(adj, x, W0, b0r, W1, b1r)

    return out


# speedup vs baseline: 1.0127x; 1.0127x over previous
import functools

import jax
import jax.numpy as jnp
from jax.experimental import pallas as pl
from jax.experimental.pallas import tpu as pltpu

_N = 10000
_BM = 80
_GM = _N // _BM
_STEPS = 2 * _GM
_NBUF = 8
_SPLITS = (0, _BM)


def _gcn_body(adj_hbm, x_ref, w0_ref, b0_ref, w1_ref, b1_ref, out_hbm,
              bufs, obuf, xw0_s, hw1_s, hw1bf_s, in_sems, out_sems):

    def adj_copies(g, slot):
        row = jax.lax.rem(g, _GM) * _BM
        return [
            pltpu.make_async_copy(
                adj_hbm.at[pl.ds(row + lo, hi - lo), :],
                bufs.at[slot, pl.ds(lo, hi - lo), :],
                in_sems.at[slot, p])
            for p, (lo, hi) in enumerate(zip(_SPLITS[:-1], _SPLITS[1:]))
        ]

    def adj_start(g, slot):
        for c in adj_copies(g, slot):
            c.start()

    def adj_wait(g, slot):
        for c in adj_copies(g, slot):
            c.wait()

    for k in range(_NBUF):
        adj_start(k, k)

    xw0_s[...] = jnp.dot(
        x_ref[...].astype(jnp.bfloat16),
        w0_ref[...].astype(jnp.bfloat16),
        preferred_element_type=jnp.float32,
    ).astype(jnp.bfloat16)
    w1b = w1_ref[...].astype(jnp.bfloat16)
    b0v = b0_ref[...]
    b1v = b1_ref[...]

    def refill(g, slot):
        @pl.when(g + _NBUF < _STEPS)
        def _():
            adj_start(g + _NBUF, slot)

    def phase0_step(i, carry):
        slot = jax.lax.rem(i, _NBUF)
        adj_wait(i, slot)
        a = bufs[slot].astype(jnp.bfloat16)
        acc = jnp.dot(a, xw0_s[...], preferred_element_type=jnp.float32)
        h = jnp.maximum(acc + b0v, 0.0)
        hw1_s[pl.ds(i * _BM, _BM), :] = jnp.dot(
            h.astype(jnp.bfloat16), w1b, preferred_element_type=jnp.float32)
        refill(i, slot)
        return carry

    jax.lax.fori_loop(0, _GM, phase0_step, 0, unroll=_NBUF)

    hw1bf_s[...] = hw1_s[...].astype(jnp.bfloat16)

    def out_copy(j, oslot):
        return pltpu.make_async_copy(
            obuf.at[oslot], out_hbm.at[pl.ds(j * _BM, _BM), :],
            out_sems.at[oslot])

    def phase1_step(j, carry):
        g = _GM + j
        slot = jax.lax.rem(g, _NBUF)
        adj_wait(g, slot)
        a = bufs[slot].astype(jnp.bfloat16)
        o = jnp.dot(a, hw1bf_s[...], preferred_element_type=jnp.float32) + b1v
        oslot = jax.lax.rem(j, 2)

        @pl.when(j >= 2)
        def _():
            out_copy(j - 2, oslot).wait()

        obuf[oslot] = o
        out_copy(j, oslot).start()
        refill(g, slot)
        return carry

    jax.lax.fori_loop(0, _GM, phase1_step, 0, unroll=_NBUF)

    out_copy(_GM - 2, jax.lax.rem(_GM - 2, 2)).wait()
    out_copy(_GM - 1, jax.lax.rem(_GM - 1, 2)).wait()


@functools.partial(jax.jit, donate_argnums=())
def kernel(x, adj, W0, b0, W1, b1):
    n, d_in = x.shape
    d_hid = W0.shape[1]
    d_out = W1.shape[1]
    b0r = b0.reshape(1, d_hid)
    b1r = b1.reshape(1, d_out)

    out = pl.pallas_call(
        _gcn_body,
        in_specs=[
            pl.BlockSpec(memory_space=pltpu.MemorySpace.HBM),
            pl.BlockSpec(memory_space=pltpu.MemorySpace.VMEM),
            pl.BlockSpec(memory_space=pltpu.MemorySpace.VMEM),
            pl.BlockSpec(memory_space=pltpu.MemorySpace.VMEM),
            pl.BlockSpec(memory_space=pltpu.MemorySpace.VMEM),
            pl.BlockSpec(memory_space=pltpu.MemorySpace.VMEM),
        ],
        out_specs=pl.BlockSpec(memory_space=pltpu.MemorySpace.HBM),
        out_shape=jax.ShapeDtypeStruct((n, d_out), jnp.float32),
        scratch_shapes=[
            pltpu.VMEM((_NBUF, _BM, _N), jnp.float32),
            pltpu.VMEM((2, _BM, 128), jnp.float32),
            pltpu.VMEM((_N, 128), jnp.bfloat16),
            pltpu.VMEM((_N, 128), jnp.float32),
            pltpu.VMEM((_N, 128), jnp.bfloat16),
            pltpu.SemaphoreType.DMA((_NBUF, len(_SPLITS) - 1)),
            pltpu.SemaphoreType.DMA((2,)),
        ],
    )(adj, x, W0, b0r, W1, b1r)

    return out


# speedup vs baseline: 1.0248x; 1.0120x over previous
import functools

import jax
import jax.numpy as jnp
from jax.experimental import pallas as pl
from jax.experimental.pallas import tpu as pltpu

_N = 10000
_BM = 80
_GM = _N // _BM
_STEPS = 2 * _GM
_NBUF = 6
_SPLITS = (0, _BM)


def _gcn_body(adj_hbm, x_ref, w0_ref, b0_ref, w1_ref, b1_ref, out_hbm,
              bufs, obuf, xw0_s, hw1_s, hw1bf_s, in_sems, out_sems):

    def adj_copies(g, slot):
        row = jax.lax.rem(g, _GM) * _BM
        return [
            pltpu.make_async_copy(
                adj_hbm.at[pl.ds(row + lo, hi - lo), :],
                bufs.at[slot, pl.ds(lo, hi - lo), :],
                in_sems.at[slot, p])
            for p, (lo, hi) in enumerate(zip(_SPLITS[:-1], _SPLITS[1:]))
        ]

    def adj_start(g, slot):
        for c in adj_copies(g, slot):
            c.start()

    def adj_wait(g, slot):
        for c in adj_copies(g, slot):
            c.wait()

    for k in range(_NBUF):
        adj_start(k, k)

    xw0_s[...] = jnp.dot(
        x_ref[...].astype(jnp.bfloat16),
        w0_ref[...].astype(jnp.bfloat16),
        preferred_element_type=jnp.float32,
    ).astype(jnp.bfloat16)
    w1b = w1_ref[...].astype(jnp.bfloat16)
    b0v = b0_ref[...]
    b1v = b1_ref[...]

    def refill(g, slot):
        @pl.when(g + _NBUF < _STEPS)
        def _():
            adj_start(g + _NBUF, slot)

    def phase0_step(i, carry):
        slot = jax.lax.rem(i, _NBUF)
        adj_wait(i, slot)
        a = bufs[slot].astype(jnp.bfloat16)
        acc = jnp.dot(a, xw0_s[...], preferred_element_type=jnp.float32)
        h = jnp.maximum(acc + b0v, 0.0)
        hw1_s[pl.ds(i * _BM, _BM), :] = jnp.dot(
            h.astype(jnp.bfloat16), w1b, preferred_element_type=jnp.float32)
        refill(i, slot)
        return carry

    jax.lax.fori_loop(0, _GM, phase0_step, 0, unroll=_NBUF)

    hw1bf_s[...] = hw1_s[...].astype(jnp.bfloat16)

    def out_copy(j, oslot):
        return pltpu.make_async_copy(
            obuf.at[oslot], out_hbm.at[pl.ds(j * _BM, _BM), :],
            out_sems.at[oslot])

    def phase1_step(j, carry):
        g = _GM + j
        slot = jax.lax.rem(g, _NBUF)
        adj_wait(g, slot)
        a = bufs[slot].astype(jnp.bfloat16)
        o = jnp.dot(a, hw1bf_s[...], preferred_element_type=jnp.float32) + b1v
        oslot = jax.lax.rem(j, 2)

        @pl.when(j >= 2)
        def _():
            out_copy(j - 2, oslot).wait()

        obuf[oslot] = o
        out_copy(j, oslot).start()
        refill(g, slot)
        return carry

    jax.lax.fori_loop(0, _GM, phase1_step, 0, unroll=_NBUF)

    out_copy(_GM - 2, jax.lax.rem(_GM - 2, 2)).wait()
    out_copy(_GM - 1, jax.lax.rem(_GM - 1, 2)).wait()


@functools.partial(jax.jit, donate_argnums=())
def kernel(x, adj, W0, b0, W1, b1):
    n, d_in = x.shape
    d_hid = W0.shape[1]
    d_out = W1.shape[1]
    b0r = b0.reshape(1, d_hid)
    b1r = b1.reshape(1, d_out)

    out = pl.pallas_call(
        _gcn_body,
        in_specs=[
            pl.BlockSpec(memory_space=pltpu.MemorySpace.HBM),
            pl.BlockSpec(memory_space=pltpu.MemorySpace.VMEM),
            pl.BlockSpec(memory_space=pltpu.MemorySpace.VMEM),
            pl.BlockSpec(memory_space=pltpu.MemorySpace.VMEM),
            pl.BlockSpec(memory_space=pltpu.MemorySpace.VMEM),
            pl.BlockSpec(memory_space=pltpu.MemorySpace.VMEM),
        ],
        out_specs=pl.BlockSpec(memory_space=pltpu.MemorySpace.HBM),
        out_shape=jax.ShapeDtypeStruct((n, d_out), jnp.float32),
        scratch_shapes=[
            pltpu.VMEM((_NBUF, _BM, _N), jnp.float32),
            pltpu.VMEM((2, _BM, 128), jnp.float32),
            pltpu.VMEM((_N, 128), jnp.bfloat16),
            pltpu.VMEM((_N, 128), jnp.float32),
            pltpu.VMEM((_N, 128), jnp.bfloat16),
            pltpu.SemaphoreType.DMA((_NBUF, len(_SPLITS) - 1)),
            pltpu.SemaphoreType.DMA((2,)),
        ],
    )(adj, x, W0, b0r, W1, b1r)

    return out


# speedup vs baseline: 1.0299x; 1.0050x over previous
import functools

import jax
import jax.numpy as jnp
from jax.experimental import pallas as pl
from jax.experimental.pallas import tpu as pltpu

_N = 10000
_BM = 80
_GM = _N // _BM
_STEPS = 2 * _GM
_NBUF = 5
_SPLITS = (0, _BM)


def _gcn_body(adj_hbm, x_ref, w0_ref, b0_ref, w1_ref, b1_ref, out_hbm,
              bufs, obuf, xw0_s, hw1_s, hw1bf_s, in_sems, out_sems):

    def adj_copies(g, slot):
        row = jax.lax.rem(g, _GM) * _BM
        return [
            pltpu.make_async_copy(
                adj_hbm.at[pl.ds(row + lo, hi - lo), :],
                bufs.at[slot, pl.ds(lo, hi - lo), :],
                in_sems.at[slot, p])
            for p, (lo, hi) in enumerate(zip(_SPLITS[:-1], _SPLITS[1:]))
        ]

    def adj_start(g, slot):
        for c in adj_copies(g, slot):
            c.start()

    def adj_wait(g, slot):
        for c in adj_copies(g, slot):
            c.wait()

    for k in range(_NBUF):
        adj_start(k, k)

    xw0_s[...] = jnp.dot(
        x_ref[...].astype(jnp.bfloat16),
        w0_ref[...].astype(jnp.bfloat16),
        preferred_element_type=jnp.float32,
    ).astype(jnp.bfloat16)
    w1b = w1_ref[...].astype(jnp.bfloat16)
    b0v = b0_ref[...]
    b1v = b1_ref[...]

    def refill(g, slot):
        @pl.when(g + _NBUF < _STEPS)
        def _():
            adj_start(g + _NBUF, slot)

    def phase0_step(i, carry):
        slot = jax.lax.rem(i, _NBUF)
        adj_wait(i, slot)
        a = bufs[slot].astype(jnp.bfloat16)
        acc = jnp.dot(a, xw0_s[...], preferred_element_type=jnp.float32)
        h = jnp.maximum(acc + b0v, 0.0)
        hw1_s[pl.ds(i * _BM, _BM), :] = jnp.dot(
            h.astype(jnp.bfloat16), w1b, preferred_element_type=jnp.float32)
        refill(i, slot)
        return carry

    jax.lax.fori_loop(0, _GM, phase0_step, 0, unroll=_NBUF)

    hw1bf_s[...] = hw1_s[...].astype(jnp.bfloat16)

    def out_copy(j, oslot):
        return pltpu.make_async_copy(
            obuf.at[oslot], out_hbm.at[pl.ds(j * _BM, _BM), :],
            out_sems.at[oslot])

    def phase1_step(j, carry):
        g = _GM + j
        slot = jax.lax.rem(g, _NBUF)
        adj_wait(g, slot)
        a = bufs[slot].astype(jnp.bfloat16)
        o = jnp.dot(a, hw1bf_s[...], preferred_element_type=jnp.float32) + b1v
        oslot = jax.lax.rem(j, 2)

        @pl.when(j >= 2)
        def _():
            out_copy(j - 2, oslot).wait()

        obuf[oslot] = o
        out_copy(j, oslot).start()
        refill(g, slot)
        return carry

    jax.lax.fori_loop(0, _GM, phase1_step, 0, unroll=_NBUF)

    out_copy(_GM - 2, jax.lax.rem(_GM - 2, 2)).wait()
    out_copy(_GM - 1, jax.lax.rem(_GM - 1, 2)).wait()


@functools.partial(jax.jit, donate_argnums=())
def kernel(x, adj, W0, b0, W1, b1):
    n, d_in = x.shape
    d_hid = W0.shape[1]
    d_out = W1.shape[1]
    b0r = b0.reshape(1, d_hid)
    b1r = b1.reshape(1, d_out)

    out = pl.pallas_call(
        _gcn_body,
        in_specs=[
            pl.BlockSpec(memory_space=pltpu.MemorySpace.HBM),
            pl.BlockSpec(memory_space=pltpu.MemorySpace.VMEM),
            pl.BlockSpec(memory_space=pltpu.MemorySpace.VMEM),
            pl.BlockSpec(memory_space=pltpu.MemorySpace.VMEM),
            pl.BlockSpec(memory_space=pltpu.MemorySpace.VMEM),
            pl.BlockSpec(memory_space=pltpu.MemorySpace.VMEM),
        ],
        out_specs=pl.BlockSpec(memory_space=pltpu.MemorySpace.HBM),
        out_shape=jax.ShapeDtypeStruct((n, d_out), jnp.float32),
        scratch_shapes=[
            pltpu.VMEM((_NBUF, _BM, _N), jnp.float32),
            pltpu.VMEM((2, _BM, 128), jnp.float32),
            pltpu.VMEM((_N, 128), jnp.bfloat16),
            pltpu.VMEM((_N, 128), jnp.float32),
            pltpu.VMEM((_N, 128), jnp.bfloat16),
            pltpu.SemaphoreType.DMA((_NBUF, len(_SPLITS) - 1)),
            pltpu.SemaphoreType.DMA((2,)),
        ],
    )(adj, x, W0, b0r, W1, b1r)

    return out


# speedup vs baseline: 1.0339x; 1.0039x over previous
import functools

import jax
import jax.numpy as jnp
from jax.experimental import pallas as pl
from jax.experimental.pallas import tpu as pltpu

_N = 10000
_BM = 80
_GM = _N // _BM
_STEPS = 2 * _GM
_NBUF = 4
_SPLITS = (0, _BM)


def _gcn_body(adj_hbm, x_ref, w0_ref, b0_ref, w1_ref, b1_ref, out_hbm,
              bufs, obuf, xw0_s, hw1_s, hw1bf_s, in_sems, out_sems):

    def adj_copies(g, slot):
        row = jax.lax.rem(g, _GM) * _BM
        return [
            pltpu.make_async_copy(
                adj_hbm.at[pl.ds(row + lo, hi - lo), :],
                bufs.at[slot, pl.ds(lo, hi - lo), :],
                in_sems.at[slot, p])
            for p, (lo, hi) in enumerate(zip(_SPLITS[:-1], _SPLITS[1:]))
        ]

    def adj_start(g, slot):
        for c in adj_copies(g, slot):
            c.start()

    def adj_wait(g, slot):
        for c in adj_copies(g, slot):
            c.wait()

    for k in range(_NBUF):
        adj_start(k, k)

    xw0_s[...] = jnp.dot(
        x_ref[...].astype(jnp.bfloat16),
        w0_ref[...].astype(jnp.bfloat16),
        preferred_element_type=jnp.float32,
    ).astype(jnp.bfloat16)
    w1b = w1_ref[...].astype(jnp.bfloat16)
    b0v = b0_ref[...]
    b1v = b1_ref[...]

    def refill(g, slot):
        @pl.when(g + _NBUF < _STEPS)
        def _():
            adj_start(g + _NBUF, slot)

    def phase0_step(i, carry):
        slot = jax.lax.rem(i, _NBUF)
        adj_wait(i, slot)
        a = bufs[slot].astype(jnp.bfloat16)
        acc = jnp.dot(a, xw0_s[...], preferred_element_type=jnp.float32)
        h = jnp.maximum(acc + b0v, 0.0)
        hw1_s[pl.ds(i * _BM, _BM), :] = jnp.dot(
            h.astype(jnp.bfloat16), w1b, preferred_element_type=jnp.float32)
        refill(i, slot)
        return carry

    jax.lax.fori_loop(0, _GM, phase0_step, 0, unroll=_NBUF)

    hw1bf_s[...] = hw1_s[...].astype(jnp.bfloat16)

    def out_copy(j, oslot):
        return pltpu.make_async_copy(
            obuf.at[oslot], out_hbm.at[pl.ds(j * _BM, _BM), :],
            out_sems.at[oslot])

    def phase1_step(j, carry):
        g = _GM + j
        slot = jax.lax.rem(g, _NBUF)
        adj_wait(g, slot)
        a = bufs[slot].astype(jnp.bfloat16)
        o = jnp.dot(a, hw1bf_s[...], preferred_element_type=jnp.float32) + b1v
        oslot = jax.lax.rem(j, 2)

        @pl.when(j >= 2)
        def _():
            out_copy(j - 2, oslot).wait()

        obuf[oslot] = o
        out_copy(j, oslot).start()
        refill(g, slot)
        return carry

    jax.lax.fori_loop(0, _GM, phase1_step, 0, unroll=_NBUF)

    out_copy(_GM - 2, jax.lax.rem(_GM - 2, 2)).wait()
    out_copy(_GM - 1, jax.lax.rem(_GM - 1, 2)).wait()


@functools.partial(jax.jit, donate_argnums=())
def kernel(x, adj, W0, b0, W1, b1):
    n, d_in = x.shape
    d_hid = W0.shape[1]
    d_out = W1.shape[1]
    b0r = b0.reshape(1, d_hid)
    b1r = b1.reshape(1, d_out)

    out = pl.pallas_call(
        _gcn_body,
        in_specs=[
            pl.BlockSpec(memory_space=pltpu.MemorySpace.HBM),
            pl.BlockSpec(memory_space=pltpu.MemorySpace.VMEM),
            pl.BlockSpec(memory_space=pltpu.MemorySpace.VMEM),
            pl.BlockSpec(memory_space=pltpu.MemorySpace.VMEM),
            pl.BlockSpec(memory_space=pltpu.MemorySpace.VMEM),
            pl.BlockSpec(memory_space=pltpu.MemorySpace.VMEM),
        ],
        out_specs=pl.BlockSpec(memory_space=pltpu.MemorySpace.HBM),
        out_shape=jax.ShapeDtypeStruct((n, d_out), jnp.float32),
        scratch_shapes=[
            pltpu.VMEM((_NBUF, _BM, _N), jnp.float32),
            pltpu.VMEM((2, _BM, 128), jnp.float32),
            pltpu.VMEM((_N, 128), jnp.bfloat16),
            pltpu.VMEM((_N, 128), jnp.float32),
            pltpu.VMEM((_N, 128), jnp.bfloat16),
            pltpu.SemaphoreType.DMA((_NBUF, len(_SPLITS) - 1)),
            pltpu.SemaphoreType.DMA((2,)),
        ],
    )(adj, x, W0, b0r, W1, b1r)

    return out
